# baseline (device time: 53623 ns/iter reference)
import jax
import jax.numpy as jnp
from jax import lax
from jax.experimental import pallas as pl
from jax.experimental.pallas import tpu as pltpu


def kernel(x, pi):
    shard_shape = x.shape

    def body(x_ref, pi_ref, out_ref, send_sem, recv_sem):
        my_x = lax.axis_index("x")
        my_y = lax.axis_index("y")
        my_z = lax.axis_index("z")
        other_x = 1 - my_x
        dst_x = pi_ref[my_x]

        barrier_sem = pltpu.get_barrier_semaphore()
        pl.semaphore_signal(
            barrier_sem,
            inc=1,
            device_id=(other_x, my_y, my_z),
            device_id_type=pl.DeviceIdType.MESH,
        )
        pl.semaphore_wait(barrier_sem, 1)

        @pl.when(dst_x == my_x)
        def _():
            out_ref[...] = x_ref[...]

        @pl.when(dst_x != my_x)
        def _():
            rdma = pltpu.make_async_remote_copy(
                src_ref=x_ref,
                dst_ref=out_ref,
                send_sem=send_sem,
                recv_sem=recv_sem,
                device_id=(dst_x, my_y, my_z),
                device_id_type=pl.DeviceIdType.MESH,
            )
            rdma.start()
            rdma.wait()

    return pl.pallas_call(
        body,
        out_shape=jax.ShapeDtypeStruct(shard_shape, jnp.float32),
        in_specs=[
            pl.BlockSpec(memory_space=pltpu.VMEM),
            pl.BlockSpec(memory_space=pltpu.SMEM),
        ],
        out_specs=pl.BlockSpec(memory_space=pltpu.VMEM),
        scratch_shapes=[
            pltpu.SemaphoreType.DMA,
            pltpu.SemaphoreType.DMA,
        ],
        compiler_params=pltpu.CompilerParams(collective_id=0),
    )(x, pi)


# device time: 36619 ns/iter; 1.4643x vs baseline; 1.4643x over previous
import jax
import jax.numpy as jnp
from jax import lax
from jax.experimental import pallas as pl
from jax.experimental.pallas import tpu as pltpu

C = 8


def kernel(x, pi):
    shard_shape = x.shape
    rows = shard_shape[1]
    half = rows // 2
    rows_per_chunk = half // C

    def body(x_ref, pi_ref, out_ref, x_send_sems, x_recv_sems, y_send_sems, y_recv_sems):
        my_x = lax.axis_index("x")
        my_y = lax.axis_index("y")
        my_z = lax.axis_index("z")
        other_x = 1 - my_x
        other_y = 1 - my_y
        dst_x = pi_ref[my_x]

        barrier_sem = pltpu.get_barrier_semaphore()
        pl.semaphore_signal(
            barrier_sem, inc=1,
            device_id=(other_x, my_y, my_z),
            device_id_type=pl.DeviceIdType.MESH,
        )
        pl.semaphore_signal(
            barrier_sem, inc=1,
            device_id=(my_x, other_y, my_z),
            device_id_type=pl.DeviceIdType.MESH,
        )
        pl.semaphore_wait(barrier_sem, 2)

        my_base = my_y * half
        other_base = other_y * half

        @pl.when(dst_x == my_x)
        def _():
            out_ref[...] = x_ref[...]

        @pl.when(dst_x != my_x)
        def _():
            x_rdmas = []
            for c in range(C):
                sl = pl.ds(my_base + c * rows_per_chunk, rows_per_chunk)
                r = pltpu.make_async_remote_copy(
                    src_ref=x_ref.at[:, sl, :],
                    dst_ref=out_ref.at[:, sl, :],
                    send_sem=x_send_sems.at[c],
                    recv_sem=x_recv_sems.at[c],
                    device_id=(other_x, my_y, my_z),
                    device_id_type=pl.DeviceIdType.MESH,
                )
                r.start()
                x_rdmas.append(r)

            y_rdmas = []
            for c in range(C):
                x_rdmas[c].wait_recv()
                sl = pl.ds(my_base + c * rows_per_chunk, rows_per_chunk)
                r = pltpu.make_async_remote_copy(
                    src_ref=out_ref.at[:, sl, :],
                    dst_ref=out_ref.at[:, sl, :],
                    send_sem=y_send_sems.at[c],
                    recv_sem=y_recv_sems.at[c],
                    device_id=(my_x, other_y, my_z),
                    device_id_type=pl.DeviceIdType.MESH,
                )
                r.start()
                y_rdmas.append(r)

            for c in range(C):
                sl = pl.ds(other_base + c * rows_per_chunk, rows_per_chunk)
                rr = pltpu.make_async_remote_copy(
                    src_ref=out_ref.at[:, sl, :],
                    dst_ref=out_ref.at[:, sl, :],
                    send_sem=y_send_sems.at[c],
                    recv_sem=y_recv_sems.at[c],
                    device_id=(my_x, other_y, my_z),
                    device_id_type=pl.DeviceIdType.MESH,
                )
                rr.wait_recv()

            for c in range(C):
                x_rdmas[c].wait_send()
                y_rdmas[c].wait_send()

    return pl.pallas_call(
        body,
        out_shape=jax.ShapeDtypeStruct(shard_shape, jnp.float32),
        in_specs=[
            pl.BlockSpec(memory_space=pltpu.VMEM),
            pl.BlockSpec(memory_space=pltpu.SMEM),
        ],
        out_specs=pl.BlockSpec(memory_space=pltpu.VMEM),
        scratch_shapes=[
            pltpu.SemaphoreType.DMA((C,)),
            pltpu.SemaphoreType.DMA((C,)),
            pltpu.SemaphoreType.DMA((C,)),
            pltpu.SemaphoreType.DMA((C,)),
        ],
        compiler_params=pltpu.CompilerParams(collective_id=0),
    )(x, pi)


# device time: 31964 ns/iter; 1.6776x vs baseline; 1.1456x over previous
import jax
import jax.numpy as jnp
from jax import lax
from jax.experimental import pallas as pl
from jax.experimental.pallas import tpu as pltpu

C = 4


def kernel(x, pi):
    shard_shape = x.shape
    rows = shard_shape[1]
    half = rows // 2
    quarter = rows // 4
    rpc = quarter // C

    def body(
        x_ref, pi_ref, out_ref,
        xs, xr,
        y1s, y1r,
        z1s, z1r,
        y2s, y2r,
        z2s, z2r,
    ):
        my_x = lax.axis_index("x")
        my_y = lax.axis_index("y")
        my_z = lax.axis_index("z")
        other_x = 1 - my_x
        other_y = 1 - my_y
        zb = lax.rem(my_z, 2)
        pair_z = my_z + 1 - 2 * zb
        dst_x = pi_ref[my_x]

        barrier_sem = pltpu.get_barrier_semaphore()
        for dev in [
            (other_x, my_y, my_z),
            (my_x, other_y, my_z),
            (my_x, my_y, pair_z),
        ]:
            pl.semaphore_signal(
                barrier_sem, inc=1,
                device_id=dev, device_id_type=pl.DeviceIdType.MESH,
            )
        pl.semaphore_wait(barrier_sem, 3)

        def sl(v, w, c):
            return pl.ds(v * half + w * quarter + c * rpc, rpc)

        def rdma(piece, c, ssem, rsem, dev, src_ref):
            s = sl(piece[0], piece[1], c)
            return pltpu.make_async_remote_copy(
                src_ref=src_ref.at[:, s, :],
                dst_ref=out_ref.at[:, s, :],
                send_sem=ssem,
                recv_sem=rsem,
                device_id=dev,
                device_id_type=pl.DeviceIdType.MESH,
            )

        x_dev = (other_x, my_y, my_z)
        y_dev = (my_x, other_y, my_z)
        z_dev = (my_x, my_y, pair_z)

        mine = (my_y, zb)
        from_y = (other_y, zb)
        from_z = (my_y, 1 - zb)
        diag = (other_y, 1 - zb)

        @pl.when(dst_x == my_x)
        def _():
            out_ref[...] = x_ref[...]

        @pl.when(dst_x != my_x)
        def _():
            x_rdmas = []
            for c in range(C):
                r = rdma(mine, c, xs.at[c], xr.at[c], x_dev, x_ref)
                r.start()
                x_rdmas.append(r)

            started = []
            for c in range(C):
                x_rdmas[c].wait_recv()
                for ssem, rsem, dev in (
                    (y1s.at[c], y1r.at[c], y_dev),
                    (z1s.at[c], z1r.at[c], z_dev),
                ):
                    r = rdma(mine, c, ssem, rsem, dev, out_ref)
                    r.start()
                    started.append(r)

            z1_rdmas = [
                rdma(from_z, c, z1s.at[c], z1r.at[c], z_dev, out_ref)
                for c in range(C)
            ]
            y1_rdmas = [
                rdma(from_y, c, y1s.at[c], y1r.at[c], y_dev, out_ref)
                for c in range(C)
            ]
            for c in range(C // 2):
                z1_rdmas[c].wait_recv()
                r = rdma(from_z, c, y2s.at[c], y2r.at[c], y_dev, out_ref)
                r.start()
                started.append(r)
            for c in range(C // 2, C):
                y1_rdmas[c].wait_recv()
                r = rdma(from_y, c, z2s.at[c - C // 2], z2r.at[c - C // 2], z_dev, out_ref)
                r.start()
                started.append(r)

            for c in range(C // 2):
                y1_rdmas[c].wait_recv()
            for c in range(C // 2, C):
                z1_rdmas[c].wait_recv()
            for c in range(C // 2):
                rdma(diag, c, y2s.at[c], y2r.at[c], y_dev, out_ref).wait_recv()
            for c in range(C // 2, C):
                rdma(diag, c, z2s.at[c - C // 2], z2r.at[c - C // 2], z_dev, out_ref).wait_recv()

            for r in x_rdmas:
                r.wait_send()
            for r in started:
                r.wait_send()

    return pl.pallas_call(
        body,
        out_shape=jax.ShapeDtypeStruct(shard_shape, jnp.float32),
        in_specs=[
            pl.BlockSpec(memory_space=pltpu.VMEM),
            pl.BlockSpec(memory_space=pltpu.SMEM),
        ],
        out_specs=pl.BlockSpec(memory_space=pltpu.VMEM),
        scratch_shapes=[
            pltpu.SemaphoreType.DMA((C,)),
            pltpu.SemaphoreType.DMA((C,)),
            pltpu.SemaphoreType.DMA((C,)),
            pltpu.SemaphoreType.DMA((C,)),
            pltpu.SemaphoreType.DMA((C,)),
            pltpu.SemaphoreType.DMA((C,)),
            pltpu.SemaphoreType.DMA((C // 2,)),
            pltpu.SemaphoreType.DMA((C // 2,)),
            pltpu.SemaphoreType.DMA((C // 2,)),
            pltpu.SemaphoreType.DMA((C // 2,)),
        ],
        compiler_params=pltpu.CompilerParams(collective_id=0),
    )(x, pi)
